# baseline (speedup 1.0000x reference)
import functools

import jax
import jax.numpy as jnp
from jax import lax
from jax.experimental import pallas as pl
from jax.experimental.pallas import tpu as pltpu
from jax.experimental.pallas import tpu_sc as plsc

D_MODEL = 768
D_FF = 3072
NE = 8
TOPK = 2
NT = 2048
NP = NT * TOPK
TILE = 128
PAD = NP + NE * TILE
NB = PAD // TILE
NGID = 48

NC, NS, L = 2, 16, 16
NW = NC * NS
TOK_W = NT // NW
SLOT_W = PAD // NW
GCHUNK = 40
VPP = NP // L


def _router_body(x_ref, g_ref, i1_ref, i2_ref, wa_ref, wb_ref, loss_ref):
    x = x_ref[...]
    logits = lax.dot_general(x, g_ref[...], (((1,), (1,)), ((), ())),
                             preferred_element_type=jnp.float32)
    m = jnp.max(logits, axis=1, keepdims=True)
    ex = jnp.exp(logits - m)
    probs = ex / jnp.sum(ex, axis=1, keepdims=True)
    iota = lax.broadcasted_iota(jnp.int32, probs.shape, 1)
    big = jnp.int32(NE)
    m1 = jnp.max(probs, axis=1, keepdims=True)
    i1 = jnp.min(jnp.where(probs == m1, iota, big), axis=1, keepdims=True)
    sel1 = iota == i1
    p2 = jnp.where(sel1, jnp.float32(-1.0), probs)
    m2 = jnp.max(p2, axis=1, keepdims=True)
    i2 = jnp.min(jnp.where(p2 == m2, iota, big), axis=1, keepdims=True)
    sel2 = iota == i2
    i1_ref[...] = i1
    i2_ref[...] = i2
    wa_ref[...] = m1
    wb_ref[...] = m2
    cnt = jnp.sum(sel1.astype(jnp.float32) + sel2.astype(jnp.float32),
                  axis=0, keepdims=True)
    psum = jnp.sum(probs, axis=0, keepdims=True)
    f_i = cnt / jnp.float32(NT * TOPK)
    p_i = psum / jnp.float32(NT)
    loss_ref[...] = jnp.sum(f_i * p_i, keepdims=True).reshape(1, 1)


def _router(x2, gate_w):
    return pl.pallas_call(
        _router_body,
        in_specs=[
            pl.BlockSpec((NT, D_MODEL), lambda: (0, 0)),
            pl.BlockSpec((NE, D_MODEL), lambda: (0, 0)),
        ],
        out_specs=[
            pl.BlockSpec((NT, 1), lambda: (0, 0)),
            pl.BlockSpec((NT, 1), lambda: (0, 0)),
            pl.BlockSpec((NT, 1), lambda: (0, 0)),
            pl.BlockSpec((NT, 1), lambda: (0, 0)),
            pl.BlockSpec((1, 1), lambda: (0, 0)),
        ],
        out_shape=[
            jax.ShapeDtypeStruct((NT, 1), jnp.int32),
            jax.ShapeDtypeStruct((NT, 1), jnp.int32),
            jax.ShapeDtypeStruct((NT, 1), jnp.float32),
            jax.ShapeDtypeStruct((NT, 1), jnp.float32),
            jax.ShapeDtypeStruct((1, 1), jnp.float32),
        ],
    )(x2, gate_w)


def _dispatch_body(i1_hbm, i2_hbm, x_hbm, xs_hbm, inva_hbm, invb_hbm,
                   gid_hbm, eids_v, rank_v, cnt_v, s_v, rids_v, gid_v,
                   rows_v, sem):
    wid = lax.axis_index("s") * NC + lax.axis_index("c")
    lanes = lax.broadcasted_iota(jnp.int32, (L,), 0)
    ones = jnp.ones((L,), jnp.int32)
    zeros = jnp.zeros((L,), jnp.int32)

    pltpu.sync_copy(i1_hbm, eids_v.at[pl.ds(0, NT)])
    pltpu.sync_copy(i2_hbm, eids_v.at[pl.ds(NT, NT)])
    cnt_v[...] = zeros

    def rank_step(i, _):
        off = pl.multiple_of(i * L, L)
        eid = eids_v[pl.ds(off, L)]
        base = plsc.load_gather(cnt_v, [eid])
        c, _unused = plsc.scan_count(eid)
        rank_v[pl.ds(off, L)] = base + c.astype(jnp.int32) - 1
        plsc.addupdate_scatter(cnt_v, [eid], ones)
        return 0

    lax.fori_loop(0, VPP, rank_step, 0)

    totals = cnt_v[...]
    padded = jnp.where(lanes < NE,
                       ((totals + (TILE - 1)) // TILE) * TILE, 0)
    prefix = plsc.cumsum(padded)
    s_v[...] = prefix
    shifted = plsc.load_gather(s_v, [jnp.maximum(lanes - 1, 0)])
    starts = jnp.where(lanes == 0, 0, shifted)
    s_v[...] = starts

    @pl.when(wid == 0)
    def _gids():
        for c in range(NGID // L):
            b = lanes + c * L
            acc = zeros
            for j in range(1, NE):
                sj = plsc.load_gather(s_v, [jnp.full((L,), j, jnp.int32)])
                acc = acc + jnp.where(b * TILE >= sj, 1, 0)
            gid_v[pl.ds(c * L, L)] = acc
        pltpu.sync_copy(gid_v, gid_hbm)

    def zero_step(i, _):
        rids_v[pl.ds(pl.multiple_of(i * L, L), L)] = zeros
        return 0

    lax.fori_loop(0, PAD // L, zero_step, 0)

    def dest_step(i, _):
        off = pl.multiple_of(i * L, L)
        eid = eids_v[pl.ds(off, L)]
        rank = rank_v[pl.ds(off, L)]
        sbase = plsc.load_gather(s_v, [eid])
        dest = sbase + rank
        rank_v[pl.ds(off, L)] = dest
        tok = (lanes + off) & (NT - 1)
        plsc.store_scatter(rids_v, [dest], tok)
        return 0

    lax.fori_loop(0, VPP, dest_step, 0)

    pltpu.sync_copy(rank_v.at[pl.ds(wid * TOK_W, TOK_W)],
                    inva_hbm.at[pl.ds(wid * TOK_W, TOK_W)])
    pltpu.sync_copy(rank_v.at[pl.ds(NT + wid * TOK_W, TOK_W)],
                    invb_hbm.at[pl.ds(wid * TOK_W, TOK_W)])

    for c in range(SLOT_W // GCHUNK):
        base = wid * SLOT_W + c * GCHUNK
        idx = rids_v.at[pl.ds(base, GCHUNK)]
        pltpu.async_copy(x_hbm.at[idx], rows_v, sem).wait()
        pltpu.sync_copy(rows_v, xs_hbm.at[pl.ds(base, GCHUNK)])


def _dispatch(i1, i2, x2):
    mesh = plsc.VectorSubcoreMesh(core_axis_name="c", subcore_axis_name="s")
    f = pl.kernel(
        _dispatch_body,
        out_type=[
            jax.ShapeDtypeStruct((PAD, D_MODEL), jnp.float32),
            jax.ShapeDtypeStruct((NT,), jnp.int32),
            jax.ShapeDtypeStruct((NT,), jnp.int32),
            jax.ShapeDtypeStruct((NGID,), jnp.int32),
        ],
        mesh=mesh,
        compiler_params=pltpu.CompilerParams(needs_layout_passes=False),
        scratch_types=[
            pltpu.VMEM((NP,), jnp.int32),
            pltpu.VMEM((NP,), jnp.int32),
            pltpu.VMEM((L,), jnp.int32),
            pltpu.VMEM((L,), jnp.int32),
            pltpu.VMEM((PAD,), jnp.int32),
            pltpu.VMEM((NGID,), jnp.int32),
            pltpu.VMEM((GCHUNK, D_MODEL), jnp.float32),
            pltpu.SemaphoreType.DMA,
        ],
    )
    return f(i1, i2, x2)


def _ffn_body(gid_ref, xs_ref, w1_ref, w2_ref, y_ref):
    xb = xs_ref[...].astype(jnp.bfloat16)
    h = lax.dot_general(xb, w1_ref[0], (((1,), (1,)), ((), ())),
                        preferred_element_type=jnp.float32)
    h = h * jax.nn.sigmoid(h)
    y_ref[...] = lax.dot_general(h.astype(jnp.bfloat16), w2_ref[0],
                                 (((1,), (1,)), ((), ())),
                                 preferred_element_type=jnp.float32)


def _ffn(gids, xs, w1b, w2b):
    grid_spec = pltpu.PrefetchScalarGridSpec(
        num_scalar_prefetch=1,
        grid=(NB,),
        in_specs=[
            pl.BlockSpec((TILE, D_MODEL), lambda b, g: (b, 0)),
            pl.BlockSpec((1, D_FF, D_MODEL), lambda b, g: (g[b], 0, 0)),
            pl.BlockSpec((1, D_MODEL, D_FF), lambda b, g: (g[b], 0, 0)),
        ],
        out_specs=pl.BlockSpec((TILE, D_MODEL), lambda b, g: (b, 0)),
    )
    return pl.pallas_call(
        _ffn_body,
        grid_spec=grid_spec,
        out_shape=jax.ShapeDtypeStruct((PAD, D_MODEL), jnp.float32),
    )(gids, xs, w1b, w2b)


def _combine_body(y_hbm, ia_hbm, ib_hbm, wa_hbm, wb_hbm, out_hbm,
                  ia_v, ib_v, wa_v, wb_v, ya_v, yb_v, o_v, sem):
    wid = lax.axis_index("s") * NC + lax.axis_index("c")
    half = TOK_W // 2
    for c in range(2):
        t0 = wid * TOK_W + c * half
        pltpu.sync_copy(ia_hbm.at[pl.ds(t0, half)], ia_v)
        pltpu.sync_copy(ib_hbm.at[pl.ds(t0, half)], ib_v)
        pltpu.sync_copy(wa_hbm.at[pl.ds(t0, half)], wa_v)
        pltpu.sync_copy(wb_hbm.at[pl.ds(t0, half)], wb_v)
        ca = pltpu.async_copy(y_hbm.at[ia_v], ya_v, sem)
        cb = pltpu.async_copy(y_hbm.at[ib_v], yb_v, sem)
        ca.wait()
        cb.wait()

        def row_step(r, _):
            sa = plsc.load_gather(wa_v, [jnp.full((L,), r, jnp.int32)])
            sb = plsc.load_gather(wb_v, [jnp.full((L,), r, jnp.int32)])
            for k in range(D_MODEL // L):
                o_v[r, pl.ds(k * L, L)] = (
                    ya_v[r, pl.ds(k * L, L)] * sa
                    + yb_v[r, pl.ds(k * L, L)] * sb)
            return 0

        lax.fori_loop(0, half, row_step, 0)
        pltpu.sync_copy(o_v, out_hbm.at[pl.ds(t0, half)])


def _combine(y, inva, invb, wa, wb):
    mesh = plsc.VectorSubcoreMesh(core_axis_name="c", subcore_axis_name="s")
    half = TOK_W // 2
    f = pl.kernel(
        _combine_body,
        out_type=jax.ShapeDtypeStruct((NT, D_MODEL), jnp.float32),
        mesh=mesh,
        compiler_params=pltpu.CompilerParams(needs_layout_passes=False),
        scratch_types=[
            pltpu.VMEM((half,), jnp.int32),
            pltpu.VMEM((half,), jnp.int32),
            pltpu.VMEM((half,), jnp.float32),
            pltpu.VMEM((half,), jnp.float32),
            pltpu.VMEM((half, D_MODEL), jnp.float32),
            pltpu.VMEM((half, D_MODEL), jnp.float32),
            pltpu.VMEM((half, D_MODEL), jnp.float32),
            pltpu.SemaphoreType.DMA,
        ],
    )
    return f(y, inva, invb, wa, wb)


def kernel(x, gate_w, w1, w2):
    B, T, D = x.shape
    x2 = x.reshape(B * T, D)
    i1, i2, wa, wb, loss = _router(x2, gate_w)
    xs, inva, invb, gids = _dispatch(i1.reshape(-1), i2.reshape(-1), x2)
    w1b = w1.astype(jnp.bfloat16)
    w2b = w2.astype(jnp.bfloat16)
    y = _ffn(gids[:NB], xs, w1b, w2b)
    out = _combine(y, inva, invb, wa.reshape(-1), wb.reshape(-1))
    return out.reshape(B, T, D), loss[0, 0]


# speedup vs baseline: 1.0291x; 1.0291x over previous
import functools

import jax
import jax.numpy as jnp
from jax import lax
from jax.experimental import pallas as pl
from jax.experimental.pallas import tpu as pltpu
from jax.experimental.pallas import tpu_sc as plsc

D_MODEL = 768
D_FF = 3072
NE = 8
TOPK = 2
NT = 2048
NP = NT * TOPK
TILE = 128
PAD = NP + NE * TILE
NB = PAD // TILE
NGID = 48

NC, NS, L = 2, 16, 16
NW = NC * NS
TOK_W = NT // NW
SLOT_W = PAD // NW
GCHUNK = 40
VPP = NP // L
UNROLL = 8


def _router_body(x_ref, g_ref, i1_ref, i2_ref, wa_ref, wb_ref, loss_ref):
    x = x_ref[...]
    logits = lax.dot_general(x, g_ref[...], (((1,), (1,)), ((), ())),
                             preferred_element_type=jnp.float32)
    m = jnp.max(logits, axis=1, keepdims=True)
    ex = jnp.exp(logits - m)
    probs = ex / jnp.sum(ex, axis=1, keepdims=True)
    iota = lax.broadcasted_iota(jnp.int32, probs.shape, 1)
    big = jnp.int32(NE)
    m1 = jnp.max(probs, axis=1, keepdims=True)
    i1 = jnp.min(jnp.where(probs == m1, iota, big), axis=1, keepdims=True)
    sel1 = iota == i1
    p2 = jnp.where(sel1, jnp.float32(-1.0), probs)
    m2 = jnp.max(p2, axis=1, keepdims=True)
    i2 = jnp.min(jnp.where(p2 == m2, iota, big), axis=1, keepdims=True)
    sel2 = iota == i2
    i1_ref[...] = i1
    i2_ref[...] = i2
    wa_ref[...] = m1
    wb_ref[...] = m2
    cnt = jnp.sum(sel1.astype(jnp.float32) + sel2.astype(jnp.float32),
                  axis=0, keepdims=True)
    psum = jnp.sum(probs, axis=0, keepdims=True)
    f_i = cnt / jnp.float32(NT * TOPK)
    p_i = psum / jnp.float32(NT)
    loss_ref[...] = jnp.sum(f_i * p_i, keepdims=True).reshape(1, 1)


def _router(x2, gate_w):
    return pl.pallas_call(
        _router_body,
        in_specs=[
            pl.BlockSpec((NT, D_MODEL), lambda: (0, 0)),
            pl.BlockSpec((NE, D_MODEL), lambda: (0, 0)),
        ],
        out_specs=[
            pl.BlockSpec((NT, 1), lambda: (0, 0)),
            pl.BlockSpec((NT, 1), lambda: (0, 0)),
            pl.BlockSpec((NT, 1), lambda: (0, 0)),
            pl.BlockSpec((NT, 1), lambda: (0, 0)),
            pl.BlockSpec((1, 1), lambda: (0, 0)),
        ],
        out_shape=[
            jax.ShapeDtypeStruct((NT, 1), jnp.int32),
            jax.ShapeDtypeStruct((NT, 1), jnp.int32),
            jax.ShapeDtypeStruct((NT, 1), jnp.float32),
            jax.ShapeDtypeStruct((NT, 1), jnp.float32),
            jax.ShapeDtypeStruct((1, 1), jnp.float32),
        ],
    )(x2, gate_w)


def _dispatch_body(i1_hbm, i2_hbm, x_hbm, zeros_hbm, xs_hbm, inva_hbm,
                   invb_hbm, gid_hbm, flg_hbm, eids_v, rank_v, cnt_v, s_v,
                   rids_v, gid_v, flg_v, rows0_v, rows1_v, sem_z, sem_g0,
                   sem_g1, sem_w0, sem_w1):
    wid = lax.axis_index("s") * NC + lax.axis_index("c")
    lanes = lax.broadcasted_iota(jnp.int32, (L,), 0)
    ones = jnp.ones((L,), jnp.int32)
    zeros = jnp.zeros((L,), jnp.int32)

    zcopy = pltpu.async_copy(zeros_hbm, rids_v, sem_z)
    pltpu.sync_copy(i1_hbm, eids_v.at[pl.ds(0, NT)])
    pltpu.sync_copy(i2_hbm, eids_v.at[pl.ds(NT, NT)])
    cnt_v[...] = zeros

    def rank_step(i, _):
        base_off = pl.multiple_of(i * (L * UNROLL), L * UNROLL)
        for j in range(UNROLL):
            off = base_off + j * L
            eid = eids_v[pl.ds(off, L)]
            base = plsc.load_gather(cnt_v, [eid])
            c, _unused = plsc.scan_count(eid)
            rank_v[pl.ds(off, L)] = base + c.astype(jnp.int32) - 1
            plsc.addupdate_scatter(cnt_v, [eid], ones)
        return 0

    lax.fori_loop(0, VPP // UNROLL, rank_step, 0)

    totals = cnt_v[...]
    padded = jnp.where(lanes < NE,
                       ((totals + (TILE - 1)) // TILE) * TILE, 0)
    prefix = plsc.cumsum(padded)
    s_v[...] = prefix
    shifted = plsc.load_gather(s_v, [jnp.maximum(lanes - 1, 0)])
    starts = jnp.where(lanes == 0, 0, shifted)
    s_v[...] = starts

    @pl.when(wid == 0)
    def _gids():
        cnt_v[...] = starts + totals
        for c in range(NGID // L):
            b = lanes + c * L
            acc = zeros
            for j in range(1, NE):
                sj = plsc.load_gather(s_v, [jnp.full((L,), j, jnp.int32)])
                acc = acc + jnp.where(b * TILE >= sj, 1, 0)
            uend = plsc.load_gather(cnt_v, [acc])
            gid_v[pl.ds(c * L, L)] = acc
            flg_v[pl.ds(c * L, L)] = jnp.where(b * TILE < uend, 1, 0)
        pltpu.sync_copy(gid_v, gid_hbm)
        pltpu.sync_copy(flg_v, flg_hbm)

    zcopy.wait()

    def dest_step(i, _):
        base_off = pl.multiple_of(i * (L * UNROLL), L * UNROLL)
        for j in range(UNROLL):
            off = base_off + j * L
            eid = eids_v[pl.ds(off, L)]
            rank = rank_v[pl.ds(off, L)]
            sbase = plsc.load_gather(s_v, [eid])
            dest = sbase + rank
            rank_v[pl.ds(off, L)] = dest
            tok = (lanes + off) & (NT - 1)
            plsc.store_scatter(rids_v, [dest], tok)
        return 0

    lax.fori_loop(0, VPP // UNROLL, dest_step, 0)

    pltpu.sync_copy(rank_v.at[pl.ds(wid * TOK_W, TOK_W)],
                    inva_hbm.at[pl.ds(wid * TOK_W, TOK_W)])
    pltpu.sync_copy(rank_v.at[pl.ds(NT + wid * TOK_W, TOK_W)],
                    invb_hbm.at[pl.ds(wid * TOK_W, TOK_W)])

    nchunk = SLOT_W // GCHUNK
    bufs = (rows0_v, rows1_v)
    gsems = (sem_g0, sem_g1)
    wsems = (sem_w0, sem_w1)

    def start_gather(c):
        base = wid * SLOT_W + c * GCHUNK
        idx = rids_v.at[pl.ds(base, GCHUNK)]
        return pltpu.async_copy(x_hbm.at[idx], bufs[c % 2], gsems[c % 2])

    gdesc = {0: start_gather(0)}
    wdesc = {}
    for c in range(nchunk):
        gdesc[c].wait()
        if c + 1 < nchunk:
            if c - 1 in wdesc:
                wdesc[c - 1].wait()
            gdesc[c + 1] = start_gather(c + 1)
        base = wid * SLOT_W + c * GCHUNK
        wdesc[c] = pltpu.async_copy(
            bufs[c % 2], xs_hbm.at[pl.ds(base, GCHUNK)], wsems[c % 2])
    wdesc[nchunk - 2].wait()
    wdesc[nchunk - 1].wait()


def _dispatch(i1, i2, x2, zeros_pad):
    mesh = plsc.VectorSubcoreMesh(core_axis_name="c", subcore_axis_name="s")
    f = pl.kernel(
        _dispatch_body,
        out_type=[
            jax.ShapeDtypeStruct((PAD, D_MODEL), jnp.float32),
            jax.ShapeDtypeStruct((NT,), jnp.int32),
            jax.ShapeDtypeStruct((NT,), jnp.int32),
            jax.ShapeDtypeStruct((NGID,), jnp.int32),
            jax.ShapeDtypeStruct((NGID,), jnp.int32),
        ],
        mesh=mesh,
        compiler_params=pltpu.CompilerParams(needs_layout_passes=False),
        scratch_types=[
            pltpu.VMEM((NP,), jnp.int32),
            pltpu.VMEM((NP,), jnp.int32),
            pltpu.VMEM((L,), jnp.int32),
            pltpu.VMEM((L,), jnp.int32),
            pltpu.VMEM((PAD,), jnp.int32),
            pltpu.VMEM((NGID,), jnp.int32),
            pltpu.VMEM((NGID,), jnp.int32),
            pltpu.VMEM((GCHUNK, D_MODEL), jnp.float32),
            pltpu.VMEM((GCHUNK, D_MODEL), jnp.float32),
            pltpu.SemaphoreType.DMA,
            pltpu.SemaphoreType.DMA,
            pltpu.SemaphoreType.DMA,
            pltpu.SemaphoreType.DMA,
            pltpu.SemaphoreType.DMA,
        ],
    )
    return f(i1, i2, x2, zeros_pad)


def _ffn_body(gid_ref, flg_ref, xs_ref, w1_ref, w2_ref, y_ref):
    b = pl.program_id(0)

    @pl.when(flg_ref[b] > 0)
    def _compute():
        xb = xs_ref[...].astype(jnp.bfloat16)
        h = lax.dot_general(xb, w1_ref[0], (((1,), (1,)), ((), ())),
                            preferred_element_type=jnp.float32)
        h = h * jax.nn.sigmoid(h)
        y_ref[...] = lax.dot_general(h.astype(jnp.bfloat16), w2_ref[0],
                                     (((1,), (1,)), ((), ())),
                                     preferred_element_type=jnp.float32)


def _ffn(gids, flags, xs, w1b, w2b):
    grid_spec = pltpu.PrefetchScalarGridSpec(
        num_scalar_prefetch=2,
        grid=(NB,),
        in_specs=[
            pl.BlockSpec((TILE, D_MODEL), lambda b, g, f: (b, 0)),
            pl.BlockSpec((1, D_FF, D_MODEL), lambda b, g, f: (g[b], 0, 0)),
            pl.BlockSpec((1, D_MODEL, D_FF), lambda b, g, f: (g[b], 0, 0)),
        ],
        out_specs=pl.BlockSpec((TILE, D_MODEL), lambda b, g, f: (b, 0)),
    )
    return pl.pallas_call(
        _ffn_body,
        grid_spec=grid_spec,
        out_shape=jax.ShapeDtypeStruct((PAD, D_MODEL), jnp.float32),
    )(gids, flags, xs, w1b, w2b)


def _combine_body(y_hbm, ia_hbm, ib_hbm, wa_hbm, wb_hbm, out_hbm,
                  ia_v, ib_v, wa_v, wb_v, ya_v, yb_v, o_v, sem):
    wid = lax.axis_index("s") * NC + lax.axis_index("c")
    half = TOK_W // 2
    for c in range(2):
        t0 = wid * TOK_W + c * half
        pltpu.sync_copy(ia_hbm.at[pl.ds(t0, half)], ia_v)
        pltpu.sync_copy(ib_hbm.at[pl.ds(t0, half)], ib_v)
        pltpu.sync_copy(wa_hbm.at[pl.ds(t0, half)], wa_v)
        pltpu.sync_copy(wb_hbm.at[pl.ds(t0, half)], wb_v)
        ca = pltpu.async_copy(y_hbm.at[ia_v], ya_v, sem)
        cb = pltpu.async_copy(y_hbm.at[ib_v], yb_v, sem)
        ca.wait()
        cb.wait()

        def row_step(r, _):
            sa = plsc.load_gather(wa_v, [jnp.full((L,), r, jnp.int32)])
            sb = plsc.load_gather(wb_v, [jnp.full((L,), r, jnp.int32)])
            for k in range(D_MODEL // L):
                o_v[r, pl.ds(k * L, L)] = (
                    ya_v[r, pl.ds(k * L, L)] * sa
                    + yb_v[r, pl.ds(k * L, L)] * sb)
            return 0

        lax.fori_loop(0, half, row_step, 0)
        pltpu.sync_copy(o_v, out_hbm.at[pl.ds(t0, half)])


def _combine(y, inva, invb, wa, wb):
    mesh = plsc.VectorSubcoreMesh(core_axis_name="c", subcore_axis_name="s")
    half = TOK_W // 2
    f = pl.kernel(
        _combine_body,
        out_type=jax.ShapeDtypeStruct((NT, D_MODEL), jnp.float32),
        mesh=mesh,
        compiler_params=pltpu.CompilerParams(needs_layout_passes=False),
        scratch_types=[
            pltpu.VMEM((half,), jnp.int32),
            pltpu.VMEM((half,), jnp.int32),
            pltpu.VMEM((half,), jnp.float32),
            pltpu.VMEM((half,), jnp.float32),
            pltpu.VMEM((half, D_MODEL), jnp.float32),
            pltpu.VMEM((half, D_MODEL), jnp.float32),
            pltpu.VMEM((half, D_MODEL), jnp.float32),
            pltpu.SemaphoreType.DMA,
        ],
    )
    return f(y, inva, invb, wa, wb)


def kernel(x, gate_w, w1, w2):
    B, T, D = x.shape
    x2 = x.reshape(B * T, D)
    i1, i2, wa, wb, loss = _router(x2, gate_w)
    zeros_pad = jnp.zeros((PAD,), jnp.int32)
    xs, inva, invb, gids, flags = _dispatch(
        i1.reshape(-1), i2.reshape(-1), x2, zeros_pad)
    w1b = w1.astype(jnp.bfloat16)
    w2b = w2.astype(jnp.bfloat16)
    y = _ffn(gids[:NB], flags[:NB], xs, w1b, w2b)
    out = _combine(y, inva, invb, wa.reshape(-1), wb.reshape(-1))
    return out.reshape(B, T, D), loss[0, 0]


# speedup vs baseline: 1.0341x; 1.0049x over previous
import functools

import jax
import jax.numpy as jnp
from jax import lax
from jax.experimental import pallas as pl
from jax.experimental.pallas import tpu as pltpu
from jax.experimental.pallas import tpu_sc as plsc

D_MODEL = 768
D_FF = 3072
NE = 8
TOPK = 2
NT = 2048
NP = NT * TOPK
TILE = 128
PAD = NP + NE * TILE
NB = PAD // TILE
NGID = 48

NC, NS, L = 2, 16, 16
NW = NC * NS
TOK_W = NT // NW
SLOT_W = PAD // NW
GCHUNK = 40
VPP = NP // L
UNROLL = 8


def _router_body(x_ref, g_ref, i1_ref, i2_ref, wa_ref, wb_ref, loss_ref):
    x = x_ref[...]
    logits = lax.dot_general(x, g_ref[...], (((1,), (1,)), ((), ())),
                             preferred_element_type=jnp.float32)
    m = jnp.max(logits, axis=1, keepdims=True)
    ex = jnp.exp(logits - m)
    probs = ex / jnp.sum(ex, axis=1, keepdims=True)
    iota = lax.broadcasted_iota(jnp.int32, probs.shape, 1)
    big = jnp.int32(NE)
    m1 = jnp.max(probs, axis=1, keepdims=True)
    i1 = jnp.min(jnp.where(probs == m1, iota, big), axis=1, keepdims=True)
    sel1 = iota == i1
    p2 = jnp.where(sel1, jnp.float32(-1.0), probs)
    m2 = jnp.max(p2, axis=1, keepdims=True)
    i2 = jnp.min(jnp.where(p2 == m2, iota, big), axis=1, keepdims=True)
    sel2 = iota == i2
    i1_ref[...] = i1
    i2_ref[...] = i2
    wa_ref[...] = m1
    wb_ref[...] = m2
    cnt = jnp.sum(sel1.astype(jnp.float32) + sel2.astype(jnp.float32),
                  axis=0, keepdims=True)
    psum = jnp.sum(probs, axis=0, keepdims=True)
    f_i = cnt / jnp.float32(NT * TOPK)
    p_i = psum / jnp.float32(NT)
    loss_ref[...] = jnp.sum(f_i * p_i, keepdims=True).reshape(1, 1)


def _router(x2, gate_w):
    return pl.pallas_call(
        _router_body,
        in_specs=[
            pl.BlockSpec((NT, D_MODEL), lambda: (0, 0)),
            pl.BlockSpec((NE, D_MODEL), lambda: (0, 0)),
        ],
        out_specs=[
            pl.BlockSpec((NT, 1), lambda: (0, 0)),
            pl.BlockSpec((NT, 1), lambda: (0, 0)),
            pl.BlockSpec((NT, 1), lambda: (0, 0)),
            pl.BlockSpec((NT, 1), lambda: (0, 0)),
            pl.BlockSpec((1, 1), lambda: (0, 0)),
        ],
        out_shape=[
            jax.ShapeDtypeStruct((NT, 1), jnp.int32),
            jax.ShapeDtypeStruct((NT, 1), jnp.int32),
            jax.ShapeDtypeStruct((NT, 1), jnp.float32),
            jax.ShapeDtypeStruct((NT, 1), jnp.float32),
            jax.ShapeDtypeStruct((1, 1), jnp.float32),
        ],
    )(x2, gate_w)


def _dispatch_body(i1_hbm, i2_hbm, x_hbm, zeros_hbm, xs_hbm, inva_hbm,
                   invb_hbm, gid_hbm, flg_hbm, eids_v, rank_v, cnt_v, s_v,
                   rids_v, gid_v, flg_v, rows0_v, rows1_v, sem_z, sem_g0,
                   sem_g1, sem_w0, sem_w1):
    wid = lax.axis_index("s") * NC + lax.axis_index("c")
    lanes = lax.broadcasted_iota(jnp.int32, (L,), 0)
    ones = jnp.ones((L,), jnp.int32)
    zeros = jnp.zeros((L,), jnp.int32)

    with jax.named_scope("disp_copyin"):
        zcopy = pltpu.async_copy(zeros_hbm, rids_v, sem_z)
        pltpu.sync_copy(i1_hbm, eids_v.at[pl.ds(0, NT)])
        pltpu.sync_copy(i2_hbm, eids_v.at[pl.ds(NT, NT)])
        cnt_v[...] = zeros

    def rank_step(i, _):
        base_off = pl.multiple_of(i * (L * UNROLL), L * UNROLL)
        for j in range(UNROLL):
            off = base_off + j * L
            eid = eids_v[pl.ds(off, L)]
            base = plsc.load_gather(cnt_v, [eid])
            c, _unused = plsc.scan_count(eid)
            rank_v[pl.ds(off, L)] = base + c.astype(jnp.int32) - 1
            plsc.addupdate_scatter(cnt_v, [eid], ones)
        return 0

    with jax.named_scope("disp_rank"):
        lax.fori_loop(0, VPP // UNROLL, rank_step, 0)

    totals = cnt_v[...]
    padded = jnp.where(lanes < NE,
                       ((totals + (TILE - 1)) // TILE) * TILE, 0)
    prefix = plsc.cumsum(padded)
    s_v[...] = prefix
    shifted = plsc.load_gather(s_v, [jnp.maximum(lanes - 1, 0)])
    starts = jnp.where(lanes == 0, 0, shifted)
    s_v[...] = starts

    @pl.when(wid == 0)
    def _gids():
        cnt_v[...] = starts + totals
        for c in range(NGID // L):
            b = lanes + c * L
            acc = zeros
            for j in range(1, NE):
                sj = plsc.load_gather(s_v, [jnp.full((L,), j, jnp.int32)])
                acc = acc + jnp.where(b * TILE >= sj, 1, 0)
            uend = plsc.load_gather(cnt_v, [acc])
            gid_v[pl.ds(c * L, L)] = acc
            flg_v[pl.ds(c * L, L)] = jnp.where(b * TILE < uend, 1, 0)
        pltpu.sync_copy(gid_v, gid_hbm)
        pltpu.sync_copy(flg_v, flg_hbm)

    zcopy.wait()

    def dest_step(i, _):
        base_off = pl.multiple_of(i * (L * UNROLL), L * UNROLL)
        for j in range(UNROLL):
            off = base_off + j * L
            eid = eids_v[pl.ds(off, L)]
            rank = rank_v[pl.ds(off, L)]
            sbase = plsc.load_gather(s_v, [eid])
            dest = sbase + rank
            rank_v[pl.ds(off, L)] = dest
            tok = (lanes + off) & (NT - 1)
            plsc.store_scatter(rids_v, [dest], tok)
        return 0

    with jax.named_scope("disp_dest"):
        lax.fori_loop(0, VPP // UNROLL, dest_step, 0)

    pltpu.sync_copy(rank_v.at[pl.ds(wid * TOK_W, TOK_W)],
                    inva_hbm.at[pl.ds(wid * TOK_W, TOK_W)])
    pltpu.sync_copy(rank_v.at[pl.ds(NT + wid * TOK_W, TOK_W)],
                    invb_hbm.at[pl.ds(wid * TOK_W, TOK_W)])

    nchunk = SLOT_W // GCHUNK
    bufs = (rows0_v, rows1_v)
    gsems = (sem_g0, sem_g1)
    wsems = (sem_w0, sem_w1)

    def start_gather(c):
        base = wid * SLOT_W + c * GCHUNK
        idx = rids_v.at[pl.ds(base, GCHUNK)]
        return pltpu.async_copy(x_hbm.at[idx], bufs[c % 2], gsems[c % 2])

    with jax.named_scope("disp_gather"):
        gdesc = {0: start_gather(0)}
        wdesc = {}
        for c in range(nchunk):
            gdesc[c].wait()
            if c + 1 < nchunk:
                if c - 1 in wdesc:
                    wdesc[c - 1].wait()
                gdesc[c + 1] = start_gather(c + 1)
            base = wid * SLOT_W + c * GCHUNK
            wdesc[c] = pltpu.async_copy(
                bufs[c % 2], xs_hbm.at[pl.ds(base, GCHUNK)], wsems[c % 2])
        wdesc[nchunk - 2].wait()
        wdesc[nchunk - 1].wait()


def _dispatch(i1, i2, x2, zeros_pad):
    mesh = plsc.VectorSubcoreMesh(core_axis_name="c", subcore_axis_name="s")
    f = pl.kernel(
        _dispatch_body,
        out_type=[
            jax.ShapeDtypeStruct((PAD, D_MODEL), jnp.float32),
            jax.ShapeDtypeStruct((NT,), jnp.int32),
            jax.ShapeDtypeStruct((NT,), jnp.int32),
            jax.ShapeDtypeStruct((NGID,), jnp.int32),
            jax.ShapeDtypeStruct((NGID,), jnp.int32),
        ],
        mesh=mesh,
        compiler_params=pltpu.CompilerParams(needs_layout_passes=False),
        scratch_types=[
            pltpu.VMEM((NP,), jnp.int32),
            pltpu.VMEM((NP,), jnp.int32),
            pltpu.VMEM((L,), jnp.int32),
            pltpu.VMEM((L,), jnp.int32),
            pltpu.VMEM((PAD,), jnp.int32),
            pltpu.VMEM((NGID,), jnp.int32),
            pltpu.VMEM((NGID,), jnp.int32),
            pltpu.VMEM((GCHUNK, D_MODEL), jnp.float32),
            pltpu.VMEM((GCHUNK, D_MODEL), jnp.float32),
            pltpu.SemaphoreType.DMA,
            pltpu.SemaphoreType.DMA,
            pltpu.SemaphoreType.DMA,
            pltpu.SemaphoreType.DMA,
            pltpu.SemaphoreType.DMA,
        ],
    )
    return f(i1, i2, x2, zeros_pad)


def _ffn_body(gid_ref, flg_ref, xs_ref, w1_ref, w2_ref, y_ref):
    b = pl.program_id(0)

    @pl.when(flg_ref[b] > 0)
    def _compute():
        xb = xs_ref[...].astype(jnp.bfloat16)
        h = lax.dot_general(xb, w1_ref[0], (((1,), (1,)), ((), ())),
                            preferred_element_type=jnp.float32)
        h = h * jax.nn.sigmoid(h)
        y_ref[...] = lax.dot_general(h.astype(jnp.bfloat16), w2_ref[0],
                                     (((1,), (1,)), ((), ())),
                                     preferred_element_type=jnp.float32)


def _ffn(gids, flags, xs, w1b, w2b):
    grid_spec = pltpu.PrefetchScalarGridSpec(
        num_scalar_prefetch=2,
        grid=(NB,),
        in_specs=[
            pl.BlockSpec((TILE, D_MODEL), lambda b, g, f: (b, 0)),
            pl.BlockSpec((1, D_FF, D_MODEL), lambda b, g, f: (g[b], 0, 0)),
            pl.BlockSpec((1, D_MODEL, D_FF), lambda b, g, f: (g[b], 0, 0)),
        ],
        out_specs=pl.BlockSpec((TILE, D_MODEL), lambda b, g, f: (b, 0)),
    )
    return pl.pallas_call(
        _ffn_body,
        grid_spec=grid_spec,
        out_shape=jax.ShapeDtypeStruct((PAD, D_MODEL), jnp.float32),
    )(gids, flags, xs, w1b, w2b)


def _combine_body(y_hbm, ia_hbm, ib_hbm, wa_hbm, wb_hbm, out_hbm,
                  ia_v, ib_v, wa_v, wb_v, ya_v, yb_v, o_v, sem):
    wid = lax.axis_index("s") * NC + lax.axis_index("c")
    half = TOK_W // 2
    for c in range(2):
        t0 = wid * TOK_W + c * half
        pltpu.sync_copy(ia_hbm.at[pl.ds(t0, half)], ia_v)
        pltpu.sync_copy(ib_hbm.at[pl.ds(t0, half)], ib_v)
        pltpu.sync_copy(wa_hbm.at[pl.ds(t0, half)], wa_v)
        pltpu.sync_copy(wb_hbm.at[pl.ds(t0, half)], wb_v)
        ca = pltpu.async_copy(y_hbm.at[ia_v], ya_v, sem)
        cb = pltpu.async_copy(y_hbm.at[ib_v], yb_v, sem)
        ca.wait()
        cb.wait()

        def row_step(r, _):
            sa = plsc.load_gather(wa_v, [jnp.full((L,), r, jnp.int32)])
            sb = plsc.load_gather(wb_v, [jnp.full((L,), r, jnp.int32)])
            for k in range(D_MODEL // L):
                o_v[r, pl.ds(k * L, L)] = (
                    ya_v[r, pl.ds(k * L, L)] * sa
                    + yb_v[r, pl.ds(k * L, L)] * sb)
            return 0

        lax.fori_loop(0, half, row_step, 0)
        pltpu.sync_copy(o_v, out_hbm.at[pl.ds(t0, half)])


def _combine(y, inva, invb, wa, wb):
    mesh = plsc.VectorSubcoreMesh(core_axis_name="c", subcore_axis_name="s")
    half = TOK_W // 2
    f = pl.kernel(
        _combine_body,
        out_type=jax.ShapeDtypeStruct((NT, D_MODEL), jnp.float32),
        mesh=mesh,
        compiler_params=pltpu.CompilerParams(needs_layout_passes=False),
        scratch_types=[
            pltpu.VMEM((half,), jnp.int32),
            pltpu.VMEM((half,), jnp.int32),
            pltpu.VMEM((half,), jnp.float32),
            pltpu.VMEM((half,), jnp.float32),
            pltpu.VMEM((half, D_MODEL), jnp.float32),
            pltpu.VMEM((half, D_MODEL), jnp.float32),
            pltpu.VMEM((half, D_MODEL), jnp.float32),
            pltpu.SemaphoreType.DMA,
        ],
    )
    return f(y, inva, invb, wa, wb)


def kernel(x, gate_w, w1, w2):
    B, T, D = x.shape
    x2 = x.reshape(B * T, D)
    i1, i2, wa, wb, loss = _router(x2, gate_w)
    zeros_pad = jnp.zeros((PAD,), jnp.int32)
    xs, inva, invb, gids, flags = _dispatch(
        i1.reshape(-1), i2.reshape(-1), x2, zeros_pad)
    w1b = w1.astype(jnp.bfloat16)
    w2b = w2.astype(jnp.bfloat16)
    y = _ffn(gids[:NB], flags[:NB], xs, w1b, w2b)
    out = _combine(y, inva, invb, wa.reshape(-1), wb.reshape(-1))
    return out.reshape(B, T, D), loss[0, 0]


# speedup vs baseline: 1.4136x; 1.3670x over previous
import functools

import jax
import jax.numpy as jnp
from jax import lax
from jax.experimental import pallas as pl
from jax.experimental.pallas import tpu as pltpu
from jax.experimental.pallas import tpu_sc as plsc

D_MODEL = 768
D_FF = 3072
NE = 8
TOPK = 2
NT = 2048
NP = NT * TOPK
TILE = 128
PAD = NP + NE * TILE
NB = PAD // TILE
NGID = 48

NC, NS, L = 2, 16, 16
NW = NC * NS
TOK_W = NT // NW
SLOT_W = PAD // NW
GCHUNK = 32
GBUFS = 4
VPP = NP // L
UNROLL = 8


def _router_body(x_ref, g_ref, i1_ref, i2_ref, wa_ref, wb_ref, loss_ref):
    x = x_ref[...]
    logits = lax.dot_general(x, g_ref[...], (((1,), (1,)), ((), ())),
                             preferred_element_type=jnp.float32)
    m = jnp.max(logits, axis=1, keepdims=True)
    ex = jnp.exp(logits - m)
    probs = ex / jnp.sum(ex, axis=1, keepdims=True)
    iota = lax.broadcasted_iota(jnp.int32, probs.shape, 1)
    big = jnp.int32(NE)
    m1 = jnp.max(probs, axis=1, keepdims=True)
    i1 = jnp.min(jnp.where(probs == m1, iota, big), axis=1, keepdims=True)
    sel1 = iota == i1
    p2 = jnp.where(sel1, jnp.float32(-1.0), probs)
    m2 = jnp.max(p2, axis=1, keepdims=True)
    i2 = jnp.min(jnp.where(p2 == m2, iota, big), axis=1, keepdims=True)
    sel2 = iota == i2
    i1_ref[...] = i1
    i2_ref[...] = i2
    wa_ref[...] = m1
    wb_ref[...] = m2
    cnt = jnp.sum(sel1.astype(jnp.float32) + sel2.astype(jnp.float32),
                  axis=0, keepdims=True)
    psum = jnp.sum(probs, axis=0, keepdims=True)
    f_i = cnt / jnp.float32(NT * TOPK)
    p_i = psum / jnp.float32(NT)
    loss_ref[...] = jnp.sum(f_i * p_i, keepdims=True).reshape(1, 1)


def _router(x2, gate_w):
    return pl.pallas_call(
        _router_body,
        in_specs=[
            pl.BlockSpec((NT, D_MODEL), lambda: (0, 0)),
            pl.BlockSpec((NE, D_MODEL), lambda: (0, 0)),
        ],
        out_specs=[
            pl.BlockSpec((NT, 1), lambda: (0, 0)),
            pl.BlockSpec((NT, 1), lambda: (0, 0)),
            pl.BlockSpec((NT, 1), lambda: (0, 0)),
            pl.BlockSpec((NT, 1), lambda: (0, 0)),
            pl.BlockSpec((1, 1), lambda: (0, 0)),
        ],
        out_shape=[
            jax.ShapeDtypeStruct((NT, 1), jnp.int32),
            jax.ShapeDtypeStruct((NT, 1), jnp.int32),
            jax.ShapeDtypeStruct((NT, 1), jnp.float32),
            jax.ShapeDtypeStruct((NT, 1), jnp.float32),
            jax.ShapeDtypeStruct((1, 1), jnp.float32),
        ],
    )(x2, gate_w)


def _dispatch_body(i1_hbm, i2_hbm, x_hbm, zeros_hbm, xs_hbm, inva_hbm,
                   invb_hbm, gid_hbm, flg_hbm, eids_v, rank_v, cnt_v, s_v,
                   rids_v, gid_v, flg_v, rows0_v, rows1_v, rows2_v, rows3_v,
                   sem_z, sem_g0, sem_g1, sem_g2, sem_g3, sem_w0, sem_w1,
                   sem_w2, sem_w3):
    wid = lax.axis_index("s") * NC + lax.axis_index("c")
    lanes = lax.broadcasted_iota(jnp.int32, (L,), 0)
    ones = jnp.ones((L,), jnp.int32)
    zeros = jnp.zeros((L,), jnp.int32)

    with jax.named_scope("disp_copyin"):
        zcopy = pltpu.async_copy(zeros_hbm, rids_v, sem_z)
        pltpu.sync_copy(i1_hbm, eids_v.at[pl.ds(0, NT)])
        pltpu.sync_copy(i2_hbm, eids_v.at[pl.ds(NT, NT)])
        cnt_v[...] = zeros

    def rank_step(i, _):
        base_off = pl.multiple_of(i * (L * UNROLL), L * UNROLL)
        for j in range(UNROLL):
            off = base_off + j * L
            eid = eids_v[pl.ds(off, L)]
            base = plsc.load_gather(cnt_v, [eid])
            c, _unused = plsc.scan_count(eid)
            rank_v[pl.ds(off, L)] = base + c.astype(jnp.int32) - 1
            plsc.addupdate_scatter(cnt_v, [eid], ones)
        return 0

    with jax.named_scope("disp_rank"):
        lax.fori_loop(0, VPP // UNROLL, rank_step, 0)

    totals = cnt_v[...]
    padded = jnp.where(lanes < NE,
                       ((totals + (TILE - 1)) // TILE) * TILE, 0)
    prefix = plsc.cumsum(padded)
    s_v[...] = prefix
    shifted = plsc.load_gather(s_v, [jnp.maximum(lanes - 1, 0)])
    starts = jnp.where(lanes == 0, 0, shifted)
    s_v[...] = starts

    @pl.when(wid == 0)
    def _gids():
        cnt_v[...] = starts + totals
        for c in range(NGID // L):
            b = lanes + c * L
            acc = zeros
            for j in range(1, NE):
                sj = plsc.load_gather(s_v, [jnp.full((L,), j, jnp.int32)])
                acc = acc + jnp.where(b * TILE >= sj, 1, 0)
            uend = plsc.load_gather(cnt_v, [acc])
            gid_v[pl.ds(c * L, L)] = acc
            flg_v[pl.ds(c * L, L)] = jnp.where(b * TILE < uend, 1, 0)
        pltpu.sync_copy(gid_v, gid_hbm)
        pltpu.sync_copy(flg_v, flg_hbm)

    zcopy.wait()

    def dest_step(i, _):
        base_off = pl.multiple_of(i * (L * UNROLL), L * UNROLL)
        for j in range(UNROLL):
            off = base_off + j * L
            eid = eids_v[pl.ds(off, L)]
            rank = rank_v[pl.ds(off, L)]
            sbase = plsc.load_gather(s_v, [eid])
            dest = sbase + rank
            rank_v[pl.ds(off, L)] = dest
            tok = (lanes + off) & (NT - 1)
            plsc.store_scatter(rids_v, [dest], tok)
        return 0

    with jax.named_scope("disp_dest"):
        lax.fori_loop(0, VPP // UNROLL, dest_step, 0)

    pltpu.sync_copy(rank_v.at[pl.ds(wid * TOK_W, TOK_W)],
                    inva_hbm.at[pl.ds(wid * TOK_W, TOK_W)])
    pltpu.sync_copy(rank_v.at[pl.ds(NT + wid * TOK_W, TOK_W)],
                    invb_hbm.at[pl.ds(wid * TOK_W, TOK_W)])

    nchunk = SLOT_W // GCHUNK
    bufs = (rows0_v, rows1_v, rows2_v, rows3_v)
    gsems = (sem_g0, sem_g1, sem_g2, sem_g3)
    wsems = (sem_w0, sem_w1, sem_w2, sem_w3)

    def start_gather(c):
        base = wid * SLOT_W + c * GCHUNK
        idx = rids_v.at[pl.ds(base, GCHUNK)]
        return pltpu.async_copy(x_hbm.at[idx], bufs[c % GBUFS],
                                gsems[c % GBUFS])

    with jax.named_scope("disp_gather"):
        gdesc = {}
        wdesc = {}
        for c in range(min(GBUFS, nchunk)):
            gdesc[c] = start_gather(c)
        for c in range(nchunk):
            gdesc[c].wait()
            base = wid * SLOT_W + c * GCHUNK
            wdesc[c] = pltpu.async_copy(
                bufs[c % GBUFS], xs_hbm.at[pl.ds(base, GCHUNK)],
                wsems[c % GBUFS])
            if c + GBUFS < nchunk:
                wdesc[c].wait()
                gdesc[c + GBUFS] = start_gather(c + GBUFS)
        for c in range(max(0, nchunk - GBUFS), nchunk):
            wdesc[c].wait()


def _dispatch(i1, i2, x2, zeros_pad):
    mesh = plsc.VectorSubcoreMesh(core_axis_name="c", subcore_axis_name="s")
    f = pl.kernel(
        _dispatch_body,
        out_type=[
            jax.ShapeDtypeStruct((PAD, D_MODEL), jnp.float32),
            jax.ShapeDtypeStruct((NT,), jnp.int32),
            jax.ShapeDtypeStruct((NT,), jnp.int32),
            jax.ShapeDtypeStruct((NGID,), jnp.int32),
            jax.ShapeDtypeStruct((NGID,), jnp.int32),
        ],
        mesh=mesh,
        compiler_params=pltpu.CompilerParams(needs_layout_passes=False),
        scratch_types=[
            pltpu.VMEM((NP,), jnp.int32),
            pltpu.VMEM((NP,), jnp.int32),
            pltpu.VMEM((L,), jnp.int32),
            pltpu.VMEM((L,), jnp.int32),
            pltpu.VMEM((PAD,), jnp.int32),
            pltpu.VMEM((NGID,), jnp.int32),
            pltpu.VMEM((NGID,), jnp.int32),
            pltpu.VMEM((GCHUNK, D_MODEL), jnp.float32),
            pltpu.VMEM((GCHUNK, D_MODEL), jnp.float32),
            pltpu.VMEM((GCHUNK, D_MODEL), jnp.float32),
            pltpu.VMEM((GCHUNK, D_MODEL), jnp.float32),
            pltpu.SemaphoreType.DMA,
            pltpu.SemaphoreType.DMA,
            pltpu.SemaphoreType.DMA,
            pltpu.SemaphoreType.DMA,
            pltpu.SemaphoreType.DMA,
            pltpu.SemaphoreType.DMA,
            pltpu.SemaphoreType.DMA,
            pltpu.SemaphoreType.DMA,
            pltpu.SemaphoreType.DMA,
        ],
    )
    return f(i1, i2, x2, zeros_pad)


def _ffn_body(gid_ref, flg_ref, xs_ref, w1_ref, w2_ref, y_ref,
              w1c_ref, w2c_ref):
    b = pl.program_id(0)
    gprev = gid_ref[jnp.maximum(b - 1, 0)]
    fresh = jnp.logical_or(b == 0, gid_ref[b] != gprev)

    @pl.when(jnp.logical_and(fresh, flg_ref[b] > 0))
    def _cast():
        w1c_ref[...] = w1_ref[0].astype(jnp.bfloat16)
        w2c_ref[...] = w2_ref[0].astype(jnp.bfloat16)

    @pl.when(flg_ref[b] > 0)
    def _compute():
        xb = xs_ref[...].astype(jnp.bfloat16)
        h = lax.dot_general(xb, w1c_ref[...], (((1,), (1,)), ((), ())),
                            preferred_element_type=jnp.float32)
        h = h * jax.nn.sigmoid(h)
        y_ref[...] = lax.dot_general(h.astype(jnp.bfloat16), w2c_ref[...],
                                     (((1,), (1,)), ((), ())),
                                     preferred_element_type=jnp.float32)


def _ffn(gids, flags, xs, w1, w2):
    grid_spec = pltpu.PrefetchScalarGridSpec(
        num_scalar_prefetch=2,
        grid=(NB,),
        in_specs=[
            pl.BlockSpec((TILE, D_MODEL), lambda b, g, f: (b, 0)),
            pl.BlockSpec((1, D_FF, D_MODEL), lambda b, g, f: (g[b], 0, 0)),
            pl.BlockSpec((1, D_MODEL, D_FF), lambda b, g, f: (g[b], 0, 0)),
        ],
        out_specs=pl.BlockSpec((TILE, D_MODEL), lambda b, g, f: (b, 0)),
        scratch_shapes=[
            pltpu.VMEM((D_FF, D_MODEL), jnp.bfloat16),
            pltpu.VMEM((D_MODEL, D_FF), jnp.bfloat16),
        ],
    )
    return pl.pallas_call(
        _ffn_body,
        grid_spec=grid_spec,
        out_shape=jax.ShapeDtypeStruct((PAD, D_MODEL), jnp.float32),
    )(gids, flags, xs, w1, w2)


def _combine_body(y_hbm, ia_hbm, ib_hbm, wa_hbm, wb_hbm, out_hbm,
                  ia_v, ib_v, wa_v, wb_v, ya_v, yb_v, o_v, sem):
    wid = lax.axis_index("s") * NC + lax.axis_index("c")
    half = TOK_W // 2
    for c in range(2):
        t0 = wid * TOK_W + c * half
        pltpu.sync_copy(ia_hbm.at[pl.ds(t0, half)], ia_v)
        pltpu.sync_copy(ib_hbm.at[pl.ds(t0, half)], ib_v)
        pltpu.sync_copy(wa_hbm.at[pl.ds(t0, half)], wa_v)
        pltpu.sync_copy(wb_hbm.at[pl.ds(t0, half)], wb_v)
        ca = pltpu.async_copy(y_hbm.at[ia_v], ya_v, sem)
        cb = pltpu.async_copy(y_hbm.at[ib_v], yb_v, sem)
        ca.wait()
        cb.wait()

        def row_step(r, _):
            sa = plsc.load_gather(wa_v, [jnp.full((L,), r, jnp.int32)])
            sb = plsc.load_gather(wb_v, [jnp.full((L,), r, jnp.int32)])
            for k in range(D_MODEL // L):
                o_v[r, pl.ds(k * L, L)] = (
                    ya_v[r, pl.ds(k * L, L)] * sa
                    + yb_v[r, pl.ds(k * L, L)] * sb)
            return 0

        lax.fori_loop(0, half, row_step, 0)
        pltpu.sync_copy(o_v, out_hbm.at[pl.ds(t0, half)])


def _combine(y, inva, invb, wa, wb):
    mesh = plsc.VectorSubcoreMesh(core_axis_name="c", subcore_axis_name="s")
    half = TOK_W // 2
    f = pl.kernel(
        _combine_body,
        out_type=jax.ShapeDtypeStruct((NT, D_MODEL), jnp.float32),
        mesh=mesh,
        compiler_params=pltpu.CompilerParams(needs_layout_passes=False),
        scratch_types=[
            pltpu.VMEM((half,), jnp.int32),
            pltpu.VMEM((half,), jnp.int32),
            pltpu.VMEM((half,), jnp.float32),
            pltpu.VMEM((half,), jnp.float32),
            pltpu.VMEM((half, D_MODEL), jnp.float32),
            pltpu.VMEM((half, D_MODEL), jnp.float32),
            pltpu.VMEM((half, D_MODEL), jnp.float32),
            pltpu.SemaphoreType.DMA,
        ],
    )
    return f(y, inva, invb, wa, wb)


def kernel(x, gate_w, w1, w2):
    B, T, D = x.shape
    x2 = x.reshape(B * T, D)
    i1, i2, wa, wb, loss = _router(x2, gate_w)
    fill_pad = jnp.arange(PAD, dtype=jnp.int32) % NT
    xs, inva, invb, gids, flags = _dispatch(
        i1.reshape(-1), i2.reshape(-1), x2, fill_pad)
    y = _ffn(gids[:NB], flags[:NB], xs, w1, w2)
    out = _combine(y, inva, invb, wa.reshape(-1), wb.reshape(-1))
    return out.reshape(B, T, D), loss[0, 0]
